# baseline (speedup 1.0000x reference)
import jax
import jax.numpy as jnp
from jax.experimental import pallas as pl
from jax.experimental.pallas import tpu as pltpu

DIM = 768
MIN_ACTIVE = 0.01
MAX_ACTIVE = 0.1


def _k_body(x_ref, w1_ref, b1_ref, w2_ref, b2_ref, k_ref):
    mean_x = jnp.mean(x_ref[...], axis=0, keepdims=True)
    h = jax.lax.dot_general(
        mean_x, w1_ref[...], (((1,), (1,)), ((), ())),
        preferred_element_type=jnp.float32,
        precision=jax.lax.Precision.DEFAULT,
    ) + b1_ref[...]
    h = 0.5 * h * (1.0 + jax.lax.erf(h * (2.0 ** -0.5)))
    z = jnp.sum(h * w2_ref[...], axis=1, keepdims=True) + b2_ref[...]
    c = jax.nn.sigmoid(z)
    ar = MIN_ACTIVE + (MAX_ACTIVE - MIN_ACTIVE) * c
    k = jnp.maximum(1, (ar * DIM).astype(jnp.int32))
    k_ref[0, 0] = k[0, 0]


def _gate_body(k_ref, x_ref, wg_ref, bgc_ref, muc_ref, scc_ref, out_ref):
    n = x_ref.shape[0]
    acc = jax.lax.dot_general(
        wg_ref[...], x_ref[...], (((1,), (1,)), ((), ())),
        preferred_element_type=jnp.float32,
        precision=jax.lax.Precision.DEFAULT,
    )
    imp_t = (jnp.abs(acc + bgc_ref[...]) - muc_ref[...]) * scc_ref[...]
    imp3 = imp_t.reshape(imp_t.shape[0] // 8, 8, n)
    kk = k_ref[0, 0]

    def finish(part):
        return jnp.broadcast_to(
            jnp.sum(part, axis=0, keepdims=True), part.shape)

    def update(cnt, cand8, t8, resolved8):
        keep = jnp.logical_or(resolved8 > 0, cnt < kk)
        t8_new = jnp.where(keep, t8, cand8)
        resolved8_new = jnp.where(cnt == kk, jnp.int32(1), resolved8)
        return t8_new, resolved8_new

    def step32(b, t8, resolved8):
        cand8 = t8 | (jnp.int32(1) << b)
        cf = jax.lax.bitcast_convert_type(cand8, jnp.float32)
        ge3 = (imp3 >= cf[None, :, :]).astype(jnp.int32)
        cnt = finish(jnp.sum(ge3, axis=0))
        return update(cnt, cand8, t8, resolved8)

    t8 = jnp.zeros((8, n), jnp.int32)
    resolved8 = jnp.zeros((8, n), jnp.int32)

    def fbody(i, carry):
        t8, resolved8 = carry
        return step32(jnp.int32(30) - i, t8, resolved8)

    t8, resolved8 = jax.lax.fori_loop(0, 7, fbody, (t8, resolved8))

    def cond(carry):
        b, t8, resolved8 = carry
        return jnp.logical_and(b >= 0, jnp.logical_not(jnp.all(resolved8 > 0)))

    def wbody(carry):
        b, t8, resolved8 = carry
        for j in range(4):
            t8, resolved8 = step32(jnp.maximum(b - j, 0), t8, resolved8)
        return b - 4, t8, resolved8

    _, t8, _ = jax.lax.while_loop(cond, wbody, (jnp.int32(23), t8, resolved8))

    impn = jnp.transpose(imp_t)
    tf = jax.lax.bitcast_convert_type(t8[0:1, :], jnp.float32)
    tcol = jnp.transpose(tf)
    out_ref[...] = jnp.where(impn >= tcol, x_ref[...], 0.0)


def kernel(x, W1, b1, W2, b2, Wg, bg, running_mean, running_var):
    B, S, D = x.shape
    H = W1.shape[0]
    xf = x.reshape(B * S, D)

    k = pl.pallas_call(
        _k_body,
        out_shape=jax.ShapeDtypeStruct((1, 1), jnp.int32),
        in_specs=[
            pl.BlockSpec((B * S, D), lambda: (0, 0)),
            pl.BlockSpec((H, D), lambda: (0, 0)),
            pl.BlockSpec((1, H), lambda: (0, 0)),
            pl.BlockSpec((1, H), lambda: (0, 0)),
            pl.BlockSpec((1, 1), lambda: (0, 0)),
        ],
        out_specs=pl.BlockSpec(memory_space=pltpu.SMEM),
    )(xf, W1, b1.reshape(1, H), W2, b2.reshape(1, 1))

    MBLK = 1024
    out = pl.pallas_call(
        _gate_body,
        out_shape=jax.ShapeDtypeStruct((B * S, D), jnp.float32),
        grid=(B * S // MBLK,),
        in_specs=[
            pl.BlockSpec(memory_space=pltpu.SMEM),
            pl.BlockSpec((MBLK, D), lambda i: (i, 0)),
            pl.BlockSpec((D, D), lambda i: (0, 0)),
            pl.BlockSpec((D, 1), lambda i: (0, 0)),
            pl.BlockSpec((D, 1), lambda i: (0, 0)),
            pl.BlockSpec((D, 1), lambda i: (0, 0)),
        ],
        out_specs=pl.BlockSpec((MBLK, D), lambda i: (i, 0)),
    )(k, xf, Wg, bg.reshape(D, 1), running_mean.reshape(D, 1),
      (1.0 / (jnp.sqrt(running_var) + 1e-06)).reshape(D, 1))
    return out.reshape(B, S, D)


# speedup vs baseline: 1.0142x; 1.0142x over previous
import jax
import jax.numpy as jnp
from jax.experimental import pallas as pl
from jax.experimental.pallas import tpu as pltpu

DIM = 768
MIN_ACTIVE = 0.01
MAX_ACTIVE = 0.1


def _k_body(x_ref, w1_ref, b1_ref, w2_ref, b2_ref, k_ref):
    mean_x = jnp.mean(x_ref[...], axis=0, keepdims=True)
    h = jax.lax.dot_general(
        mean_x, w1_ref[...], (((1,), (1,)), ((), ())),
        preferred_element_type=jnp.float32,
        precision=jax.lax.Precision.DEFAULT,
    ) + b1_ref[...]
    h = 0.5 * h * (1.0 + jax.lax.erf(h * (2.0 ** -0.5)))
    z = jnp.sum(h * w2_ref[...], axis=1, keepdims=True) + b2_ref[...]
    c = jax.nn.sigmoid(z)
    ar = MIN_ACTIVE + (MAX_ACTIVE - MIN_ACTIVE) * c
    k = jnp.maximum(1, (ar * DIM).astype(jnp.int32))
    k_ref[0, 0] = k[0, 0]


def _gate_body(k_ref, x_ref, wg_ref, bgc_ref, muc_ref, scc_ref, out_ref):
    n = x_ref.shape[0]
    acc = jax.lax.dot_general(
        wg_ref[...], x_ref[...], (((1,), (1,)), ((), ())),
        preferred_element_type=jnp.float32,
        precision=jax.lax.Precision.DEFAULT,
    )
    imp_t = (jnp.abs(acc + bgc_ref[...]) - muc_ref[...]) * scc_ref[...]
    imp3 = imp_t.reshape(imp_t.shape[0] // 8, 8, n)
    kk = k_ref[0, 0]

    def finish(part):
        return jnp.broadcast_to(
            jnp.sum(part, axis=0, keepdims=True), part.shape)

    def update(cnt, cand8, t8, resolved8):
        keep = jnp.logical_or(resolved8 > 0, cnt < kk)
        t8_new = jnp.where(keep, t8, cand8)
        resolved8_new = jnp.where(cnt == kk, jnp.int32(1), resolved8)
        return t8_new, resolved8_new

    def step32(b, t8, resolved8):
        cand8 = t8 | (jnp.int32(1) << b)
        cf = jax.lax.bitcast_convert_type(cand8, jnp.float32)
        ge3 = (imp3 >= cf[None, :, :]).astype(jnp.int32)
        cnt = finish(jnp.sum(ge3, axis=0))
        return update(cnt, cand8, t8, resolved8)

    t8 = jnp.zeros((8, n), jnp.int32)
    resolved8 = jnp.zeros((8, n), jnp.int32)

    def fbody(i, carry):
        t8, resolved8 = carry
        return step32(jnp.int32(30) - i, t8, resolved8)

    t8, resolved8 = jax.lax.fori_loop(0, 15, fbody, (t8, resolved8))

    def cond(carry):
        b, t8, resolved8 = carry
        return jnp.logical_and(b >= 0, jnp.logical_not(jnp.all(resolved8 > 0)))

    def wbody(carry):
        b, t8, resolved8 = carry
        for j in range(4):
            t8, resolved8 = step32(jnp.maximum(b - j, 0), t8, resolved8)
        return b - 4, t8, resolved8

    _, t8, _ = jax.lax.while_loop(cond, wbody, (jnp.int32(15), t8, resolved8))

    impn = jnp.transpose(imp_t)
    tf = jax.lax.bitcast_convert_type(t8[0:1, :], jnp.float32)
    tcol = jnp.transpose(tf)
    out_ref[...] = jnp.where(impn >= tcol, x_ref[...], 0.0)


def kernel(x, W1, b1, W2, b2, Wg, bg, running_mean, running_var):
    B, S, D = x.shape
    H = W1.shape[0]
    xf = x.reshape(B * S, D)

    k = pl.pallas_call(
        _k_body,
        out_shape=jax.ShapeDtypeStruct((1, 1), jnp.int32),
        in_specs=[
            pl.BlockSpec((B * S, D), lambda: (0, 0)),
            pl.BlockSpec((H, D), lambda: (0, 0)),
            pl.BlockSpec((1, H), lambda: (0, 0)),
            pl.BlockSpec((1, H), lambda: (0, 0)),
            pl.BlockSpec((1, 1), lambda: (0, 0)),
        ],
        out_specs=pl.BlockSpec(memory_space=pltpu.SMEM),
    )(xf, W1, b1.reshape(1, H), W2, b2.reshape(1, 1))

    MBLK = 1024
    out = pl.pallas_call(
        _gate_body,
        out_shape=jax.ShapeDtypeStruct((B * S, D), jnp.float32),
        grid=(B * S // MBLK,),
        in_specs=[
            pl.BlockSpec(memory_space=pltpu.SMEM),
            pl.BlockSpec((MBLK, D), lambda i: (i, 0)),
            pl.BlockSpec((D, D), lambda i: (0, 0)),
            pl.BlockSpec((D, 1), lambda i: (0, 0)),
            pl.BlockSpec((D, 1), lambda i: (0, 0)),
            pl.BlockSpec((D, 1), lambda i: (0, 0)),
        ],
        out_specs=pl.BlockSpec((MBLK, D), lambda i: (i, 0)),
    )(k, xf, Wg, bg.reshape(D, 1), running_mean.reshape(D, 1),
      (1.0 / (jnp.sqrt(running_var) + 1e-06)).reshape(D, 1))
    return out.reshape(B, S, D)


# speedup vs baseline: 1.0285x; 1.0141x over previous
import jax
import jax.numpy as jnp
from jax.experimental import pallas as pl
from jax.experimental.pallas import tpu as pltpu

DIM = 768
MIN_ACTIVE = 0.01
MAX_ACTIVE = 0.1


def _k_body(x_ref, w1_ref, b1_ref, w2_ref, b2_ref, k_ref):
    mean_x = jnp.mean(x_ref[...], axis=0, keepdims=True)
    h = jax.lax.dot_general(
        mean_x, w1_ref[...], (((1,), (1,)), ((), ())),
        preferred_element_type=jnp.float32,
        precision=jax.lax.Precision.DEFAULT,
    ) + b1_ref[...]
    h = 0.5 * h * (1.0 + jax.lax.erf(h * (2.0 ** -0.5)))
    z = jnp.sum(h * w2_ref[...], axis=1, keepdims=True) + b2_ref[...]
    c = jax.nn.sigmoid(z)
    ar = MIN_ACTIVE + (MAX_ACTIVE - MIN_ACTIVE) * c
    k = jnp.maximum(1, (ar * DIM).astype(jnp.int32))
    k_ref[0, 0] = k[0, 0]


def _gate_body(k_ref, x_ref, wg_ref, bgc_ref, muc_ref, scc_ref, out_ref):
    n = x_ref.shape[0]
    acc = jax.lax.dot_general(
        wg_ref[...], x_ref[...], (((1,), (1,)), ((), ())),
        preferred_element_type=jnp.float32,
        precision=jax.lax.Precision.DEFAULT,
    )
    imp_t = (jnp.abs(acc + bgc_ref[...]) - muc_ref[...]) * scc_ref[...]
    imp3 = imp_t.reshape(imp_t.shape[0] // 8, 8, n)
    kk = k_ref[0, 0]

    def finish(part):
        return jnp.broadcast_to(
            jnp.sum(part, axis=0, keepdims=True), part.shape)

    def update(cnt, cand8, t8, resolved8):
        keep = jnp.logical_or(resolved8 > 0, cnt < kk)
        t8_new = jnp.where(keep, t8, cand8)
        resolved8_new = jnp.where(cnt == kk, jnp.int32(1), resolved8)
        return t8_new, resolved8_new

    def step32(b, t8, resolved8):
        cand8 = t8 | (jnp.int32(1) << b)
        cf = jax.lax.bitcast_convert_type(cand8, jnp.float32)
        ge3 = (imp3 >= cf[None, :, :]).astype(jnp.int32)
        cnt = finish(jnp.sum(ge3, axis=0))
        return update(cnt, cand8, t8, resolved8)

    t8 = jnp.zeros((8, n), jnp.int32)
    resolved8 = jnp.zeros((8, n), jnp.int32)

    def fbody(i, carry):
        t8, resolved8 = carry
        return step32(jnp.int32(30) - i, t8, resolved8)

    t8, resolved8 = jax.lax.fori_loop(0, 23, fbody, (t8, resolved8))

    def cond(carry):
        b, t8, resolved8 = carry
        return jnp.logical_and(b >= 0, jnp.logical_not(jnp.all(resolved8 > 0)))

    def wbody(carry):
        b, t8, resolved8 = carry
        for j in range(4):
            t8, resolved8 = step32(jnp.maximum(b - j, 0), t8, resolved8)
        return b - 4, t8, resolved8

    _, t8, _ = jax.lax.while_loop(cond, wbody, (jnp.int32(7), t8, resolved8))

    impn = jnp.transpose(imp_t)
    tf = jax.lax.bitcast_convert_type(t8[0:1, :], jnp.float32)
    tcol = jnp.transpose(tf)
    out_ref[...] = jnp.where(impn >= tcol, x_ref[...], 0.0)


def kernel(x, W1, b1, W2, b2, Wg, bg, running_mean, running_var):
    B, S, D = x.shape
    H = W1.shape[0]
    xf = x.reshape(B * S, D)

    k = pl.pallas_call(
        _k_body,
        out_shape=jax.ShapeDtypeStruct((1, 1), jnp.int32),
        in_specs=[
            pl.BlockSpec((B * S, D), lambda: (0, 0)),
            pl.BlockSpec((H, D), lambda: (0, 0)),
            pl.BlockSpec((1, H), lambda: (0, 0)),
            pl.BlockSpec((1, H), lambda: (0, 0)),
            pl.BlockSpec((1, 1), lambda: (0, 0)),
        ],
        out_specs=pl.BlockSpec(memory_space=pltpu.SMEM),
    )(xf, W1, b1.reshape(1, H), W2, b2.reshape(1, 1))

    MBLK = 1024
    out = pl.pallas_call(
        _gate_body,
        out_shape=jax.ShapeDtypeStruct((B * S, D), jnp.float32),
        grid=(B * S // MBLK,),
        in_specs=[
            pl.BlockSpec(memory_space=pltpu.SMEM),
            pl.BlockSpec((MBLK, D), lambda i: (i, 0)),
            pl.BlockSpec((D, D), lambda i: (0, 0)),
            pl.BlockSpec((D, 1), lambda i: (0, 0)),
            pl.BlockSpec((D, 1), lambda i: (0, 0)),
            pl.BlockSpec((D, 1), lambda i: (0, 0)),
        ],
        out_specs=pl.BlockSpec((MBLK, D), lambda i: (i, 0)),
    )(k, xf, Wg, bg.reshape(D, 1), running_mean.reshape(D, 1),
      (1.0 / (jnp.sqrt(running_var) + 1e-06)).reshape(D, 1))
    return out.reshape(B, S, D)
